# Initial kernel scaffold; baseline (speedup 1.0000x reference)
import jax
import jax.numpy as jnp
from jax.experimental import pallas as pl


def kernel(edge_index, emb_weight, alpha):
    raise NotImplementedError("write your pallas kernel here")


# speedup vs baseline: 5.1821x; 5.1821x over previous
import functools

import jax
import jax.numpy as jnp
from jax import lax
from jax.experimental import pallas as pl
from jax.experimental.pallas import tpu as pltpu
from jax.experimental.pallas import tpu_sc as plsc

N_NODES = 50000
HALF = 25000
D = 64
E = 800000
NUM_LAYERS = 3

NC = 2
NS = 16
L = 16

TRASH = HALF
ACC_ROWS = 25008
NZCHUNK = 49

EB = 128
EPT = E // NS
NFULL = EPT // EB
TAIL = EPT - NFULL * EB

EPW = E // (NC * NS)
NFULL2 = EPW // EB
TAIL2 = EPW - NFULL2 * EB

_mesh = plsc.VectorSubcoreMesh(core_axis_name="c", subcore_axis_name="s")


def _zero_block(ref, rows):
    z = jnp.zeros((L,), jnp.float32)

    def body(r, _):
        for k in range(D // L):
            ref[r, pl.ds(k * L, L)] = z
        return 0

    lax.fori_loop(0, rows, body, 0)


def _fill_ones(ref, rows):
    o = jnp.full((L,), 1.0, jnp.float32)

    def body(r, _):
        for k in range(D // L):
            ref[r, pl.ds(k * L, L)] = o
        return 0

    lax.fori_loop(0, rows, body, 0)


def _make_propagate(gather: bool):
    scratch = [
        pltpu.VMEM((EB,), jnp.int32),
        pltpu.VMEM((TAIL,), jnp.int32),
        pltpu.VMEM((EB, D), jnp.float32),
        pltpu.VMEM((TAIL, D), jnp.float32),
        pltpu.VMEM((32, D), jnp.float32),
        pltpu.VMEM_SHARED((ACC_ROWS, D), jnp.float32),
        pltpu.SemaphoreType.DMA,
    ]
    if gather:
        scratch = [pltpu.VMEM((EB,), jnp.int32),
                   pltpu.VMEM((TAIL,), jnp.int32)] + scratch

    def body(*refs):
        if gather:
            (y_hbm, row_hbm, colloc_hbm, acc_hbm,
             irow, irow_t, icol, icol_t, rows, rows_t, zblk, acc, sem) = refs
        else:
            (colloc_hbm, acc_hbm,
             icol, icol_t, rows, rows_t, zblk, acc, sem) = refs

        c = lax.axis_index("c")
        s = lax.axis_index("s")

        _zero_block(zblk, 32)
        if not gather:
            _fill_ones(rows, EB)
            _fill_ones(rows_t, TAIL)

        def zbody(i, _):
            chunk = s * NZCHUNK + i

            @pl.when(chunk < HALF // 32)
            def _():
                pltpu.sync_copy(zblk, acc.at[pl.ds(chunk * 32, 32)])

            @pl.when(chunk == HALF // 32)
            def _():
                pltpu.sync_copy(zblk.at[pl.ds(0, 8)],
                                acc.at[pl.ds((HALF // 32) * 32, 8)])

            return 0

        lax.fori_loop(0, NZCHUNK, zbody, 0)
        plsc.subcore_barrier()

        base_e = s * EPT
        cbase = c * E

        def ebody(i, _):
            off = base_e + i * EB
            pltpu.sync_copy(colloc_hbm.at[pl.ds(cbase + off, EB)], icol)
            if gather:
                pltpu.sync_copy(row_hbm.at[pl.ds(off, EB)], irow)
                pltpu.async_copy(y_hbm.at[irow], rows, sem).wait()
            pltpu.sync_copy(rows, acc.at[icol], add=True)
            return 0

        lax.fori_loop(0, NFULL, ebody, 0)

        off = base_e + NFULL * EB
        pltpu.sync_copy(colloc_hbm.at[pl.ds(cbase + off, TAIL)], icol_t)
        if gather:
            pltpu.sync_copy(row_hbm.at[pl.ds(off, TAIL)], irow_t)
            pltpu.async_copy(y_hbm.at[irow_t], rows_t, sem).wait()
        pltpu.sync_copy(rows_t, acc.at[icol_t], add=True)

        plsc.subcore_barrier()

        cw = 1568
        row0 = s * cw

        @pl.when(s < NS - 1)
        def _():
            pltpu.sync_copy(acc.at[pl.ds(row0, cw)],
                            acc_hbm.at[pl.ds(c * HALF + row0, cw)])

        @pl.when(s == NS - 1)
        def _():
            last = HALF - (NS - 1) * cw
            pltpu.sync_copy(acc.at[pl.ds(row0, last)],
                            acc_hbm.at[pl.ds(c * HALF + row0, last)])

    return pl.kernel(
        body,
        out_type=jax.ShapeDtypeStruct((N_NODES, D), jnp.float32),
        mesh=_mesh, scratch_types=scratch,
        compiler_params=pltpu.CompilerParams(use_tc_tiling_on_sc=False))


_prop_gather = _make_propagate(True)
_prop_ones = _make_propagate(False)


def _edge_dot_body(out_tab, row_hbm, col_hbm, res_hbm,
                   ia, ib, ia_t, ib_t, abuf, bbuf, abuf_t, bbuf_t,
                   pbuf, pbuf_t, sem):
    c = lax.axis_index("c")
    s = lax.axis_index("s")
    wid = s * NC + c
    base_e = wid * EPW

    def batch(off, iaref, ibref, aref, bref, pref, bsize):
        pltpu.sync_copy(row_hbm.at[pl.ds(off, bsize)], iaref)
        pltpu.sync_copy(col_hbm.at[pl.ds(off, bsize)], ibref)
        pltpu.async_copy(out_tab.at[iaref], aref, sem).wait()
        pltpu.async_copy(out_tab.at[ibref], bref, sem).wait()

        def dot1(e, _):
            p = aref[e, pl.ds(0, L)] * bref[e, pl.ds(0, L)]
            for k in range(1, D // L):
                p = p + aref[e, pl.ds(k * L, L)] * bref[e, pl.ds(k * L, L)]
            pref[e] = p
            return 0

        lax.fori_loop(0, bsize, dot1, 0)
        pltpu.sync_copy(pref, res_hbm.at[pl.ds(off, bsize)])

    def ebody(i, _):
        batch(base_e + i * EB, ia, ib, abuf, bbuf, pbuf, EB)
        return 0

    lax.fori_loop(0, NFULL2, ebody, 0)
    batch(base_e + NFULL2 * EB, ia_t, ib_t, abuf_t, bbuf_t, pbuf_t, TAIL2)


_edge_dot = pl.kernel(
    _edge_dot_body,
    out_type=jax.ShapeDtypeStruct((E, L), jnp.float32),
    mesh=_mesh,
    scratch_types=[
        pltpu.VMEM((EB,), jnp.int32),
        pltpu.VMEM((EB,), jnp.int32),
        pltpu.VMEM((TAIL2,), jnp.int32),
        pltpu.VMEM((TAIL2,), jnp.int32),
        pltpu.VMEM((EB, D), jnp.float32),
        pltpu.VMEM((EB, D), jnp.float32),
        pltpu.VMEM((TAIL2, D), jnp.float32),
        pltpu.VMEM((TAIL2, D), jnp.float32),
        pltpu.VMEM((EB, L), jnp.float32),
        pltpu.VMEM((TAIL2, L), jnp.float32),
        pltpu.SemaphoreType.DMA,
    ],
    compiler_params=pltpu.CompilerParams(use_tc_tiling_on_sc=False))


_RBLK = 4000


def _reduce_body(p_ref, res_ref):
    res_ref[...] = jnp.sum(p_ref[...], axis=1, keepdims=True)


def _tc_reduce(p):
    res = pl.pallas_call(
        _reduce_body,
        grid=(E // _RBLK,),
        in_specs=[pl.BlockSpec((_RBLK, L), lambda i: (i, 0))],
        out_specs=pl.BlockSpec((_RBLK, 1), lambda i: (i, 0)),
        out_shape=jax.ShapeDtypeStruct((E, 1), jnp.float32),
    )(p)
    return res.reshape(E)


_COL_ROWS = E // 128
_CBLK = 6250


def _colloc_body(col_ref, c0_ref, c1_ref):
    col = col_ref[...]
    c0_ref[...] = jnp.where(col < HALF, col, TRASH)
    c1_ref[...] = jnp.where(col >= HALF, col - HALF, TRASH)


def _tc_colloc(col):
    col2 = col.reshape(_COL_ROWS, 128)
    c0, c1 = pl.pallas_call(
        _colloc_body,
        grid=(_COL_ROWS // _CBLK,),
        in_specs=[pl.BlockSpec((_CBLK, 128), lambda i: (i, 0))],
        out_specs=[pl.BlockSpec((_CBLK, 128), lambda i: (i, 0))] * 2,
        out_shape=[jax.ShapeDtypeStruct((_COL_ROWS, 128), jnp.int32)] * 2,
    )(col2)
    return jnp.concatenate([c0.reshape(E), c1.reshape(E)])


_NBLK = 2000


def _init_body(deg_ref, emb_ref, a0_ref, dis_ref, y_ref, out_ref):
    deg = deg_ref[...]
    dis = jnp.where(deg > 0, lax.rsqrt(deg), 0.0)
    dis_ref[...] = dis
    y_ref[...] = dis * emb_ref[...]
    out_ref[...] = a0_ref[0, 0] * emb_ref[...]


def _tc_init(deg_rep, emb, alpha0):
    return pl.pallas_call(
        _init_body,
        grid=(N_NODES // _NBLK,),
        in_specs=[pl.BlockSpec((_NBLK, D), lambda i: (i, 0)),
                  pl.BlockSpec((_NBLK, D), lambda i: (i, 0)),
                  pl.BlockSpec(memory_space=pltpu.SMEM)],
        out_specs=[pl.BlockSpec((_NBLK, D), lambda i: (i, 0))] * 3,
        out_shape=[jax.ShapeDtypeStruct((N_NODES, D), jnp.float32)] * 3,
    )(deg_rep, emb, alpha0.reshape(1, 1))


def _layer_body(acc_ref, dis_ref, out_ref, al_ref, out2_ref, y_ref):
    dis = dis_ref[...]
    x = dis * acc_ref[...]
    out2_ref[...] = out_ref[...] + al_ref[0, 0] * x
    y_ref[...] = dis * x


def _tc_layer(acc, dis, out, alpha_l):
    return pl.pallas_call(
        _layer_body,
        grid=(N_NODES // _NBLK,),
        in_specs=[pl.BlockSpec((_NBLK, D), lambda i: (i, 0)),
                  pl.BlockSpec((_NBLK, D), lambda i: (i, 0)),
                  pl.BlockSpec((_NBLK, D), lambda i: (i, 0)),
                  pl.BlockSpec(memory_space=pltpu.SMEM)],
        out_specs=[pl.BlockSpec((_NBLK, D), lambda i: (i, 0))] * 2,
        out_shape=[jax.ShapeDtypeStruct((N_NODES, D), jnp.float32)] * 2,
    )(acc, dis, out, alpha_l.reshape(1, 1))


def kernel(edge_index, emb_weight, alpha):
    row = edge_index[0]
    col = edge_index[1]
    colloc = _tc_colloc(col)

    deg_rep = _prop_ones(colloc)
    dis, y, out = _tc_init(deg_rep, emb_weight, alpha[0])

    for l in range(NUM_LAYERS):
        acc = _prop_gather(y, row, colloc)
        out, y = _tc_layer(acc, dis, out, alpha[l + 1])

    return _tc_reduce(_edge_dot(out, row, col))


# speedup vs baseline: 9.7095x; 1.8737x over previous
import jax
import jax.numpy as jnp
from jax import lax
from jax.experimental import pallas as pl
from jax.experimental.pallas import tpu as pltpu
from jax.experimental.pallas import tpu_sc as plsc

N_NODES = 50000
D = 64
DH = 32
E = 800000
NUM_LAYERS = 3

NC = 2
NS = 16
L = 16

TRASH = N_NODES
ACC_ROWS = 50008

EB = 128
RING = 3

NBG = 396
EPT_G = NBG * EB

NBO = 198
EPT_O = NBO * EB

NBD = 196
EPW = E // (NC * NS)
DLAST = EPW - (NBD - 1) * EB

_mesh = plsc.VectorSubcoreMesh(core_axis_name="c", subcore_axis_name="s")
_sc_params = pltpu.CompilerParams(use_tc_tiling_on_sc=False)


def _fill(ref, rows, width, val):
    v = jnp.full((L,), val, jnp.float32)

    def body(r, _):
        for k in range(width // L):
            ref[r, pl.ds(k * L, L)] = v
        return 0

    lax.fori_loop(0, rows, body, 0)


def _make_propagate(gather: bool):
    nb = NBG if gather else NBO
    ept = nb * EB
    scratch = [pltpu.VMEM((EB,), jnp.int32) for _ in range(6)]
    if gather:
        scratch += [pltpu.VMEM((EB,), jnp.int32) for _ in range(6)]
        scratch += [pltpu.VMEM((EB, DH), jnp.float32)] * RING
    else:
        scratch += [pltpu.VMEM((EB, DH), jnp.float32)]
    scratch += [
        pltpu.VMEM((16, DH), jnp.float32),
        pltpu.VMEM_SHARED((ACC_ROWS, DH), jnp.float32),
    ]
    scratch += [pltpu.SemaphoreType.DMA] * 6
    scratch += [pltpu.SemaphoreType.DMA] * (2 * RING if gather else RING)

    def body(*refs):
        if gather:
            (y_hbm, rowp_hbm, colp_hbm, acc_hbm) = refs[:4]
            icol = list(refs[4:10])
            irow = list(refs[10:16])
            rows = list(refs[16:19])
            zblk, acc = refs[19:21]
            isem = list(refs[21:27])
            gsem = list(refs[27:30])
            ssem = list(refs[30:33])
        else:
            (colp_hbm, acc_hbm) = refs[:2]
            icol = list(refs[2:8])
            ones = refs[8]
            zblk, acc = refs[9:11]
            isem = list(refs[11:17])
            ssem = list(refs[17:20])

        c = lax.axis_index("c")
        s = lax.axis_index("s")

        if not gather:
            _fill(ones, EB, DH, 1.0)

        _fill(zblk, 16, DH, 0.0)
        nchunk = N_NODES // 16
        per_tile = (nchunk + NS - 1) // NS

        def zbody(i, _):
            chunk = s * per_tile + i

            @pl.when(chunk < nchunk)
            def _():
                pltpu.sync_copy(zblk, acc.at[pl.ds(chunk * 16, 16)])

            return 0

        lax.fori_loop(0, per_tile, zbody, 0)
        plsc.subcore_barrier()

        def off(j):
            return c * (NS * ept) + s * ept + j * EB

        def start_i(j, m):
            pltpu.async_copy(colp_hbm.at[pl.ds(off(j), EB)], icol[m],
                             isem[m])
            if gather:
                pltpu.async_copy(rowp_hbm.at[pl.ds(off(j), EB)], irow[m],
                                 isem[m])

        def wait_i(j, m):
            pltpu.make_async_copy(colp_hbm.at[pl.ds(off(j), EB)], icol[m],
                                  isem[m]).wait()
            if gather:
                pltpu.make_async_copy(rowp_hbm.at[pl.ds(off(j), EB)],
                                      irow[m], isem[m]).wait()

        def start_s(j, k, m):
            src = rows[k] if gather else ones
            pltpu.async_copy(src, acc.at[icol[m]], ssem[k], add=True)

        def wait_s(j, k, m):
            src = rows[k] if gather else ones
            pltpu.make_async_copy(src, acc.at[icol[m]], ssem[k]).wait()

        def start_g(j, k, m):
            pltpu.async_copy(y_hbm.at[irow[m]], rows[k], gsem[k])

        def wait_g(j, k, m):
            pltpu.make_async_copy(y_hbm.at[irow[m]], rows[k],
                                  gsem[k]).wait()

        for m in range(6):
            start_i(m, m)

        def group(i, _):
            for g in range(2):
                for k in range(RING):
                    j = 6 * i + 3 * g + k
                    m = 3 * g + k
                    mp = (m + 3) % 6

                    @pl.when(j >= RING)
                    def _():
                        wait_s(j - RING, k, mp)

                        @pl.when(j + RING < nb)
                        def _():
                            start_i(jnp.minimum(j + RING, nb - 1), mp)

                    wait_i(j, m)
                    if gather:
                        start_g(j, k, m)
                if gather:
                    for k in range(RING):
                        j = 6 * i + 3 * g + k
                        m = 3 * g + k
                        wait_g(j, k, m)
                        start_s(j, k, m)
                else:
                    for k in range(RING):
                        j = 6 * i + 3 * g + k
                        m = 3 * g + k
                        start_s(j, k, m)
            return 0

        lax.fori_loop(0, nb // 6, group, 0)
        for k in range(RING):
            wait_s(nb - RING + k, k, 3 + k)

        plsc.subcore_barrier()

        cw = N_NODES // NS
        row0 = s * cw
        pltpu.sync_copy(acc.at[pl.ds(row0, cw)],
                        acc_hbm.at[pl.ds(c * N_NODES + row0, cw)])

    return pl.kernel(
        body,
        out_type=jax.ShapeDtypeStruct((NC * N_NODES, DH), jnp.float32),
        mesh=_mesh, scratch_types=scratch,
        compiler_params=_sc_params)


_prop_gather = _make_propagate(True)
_prop_ones = _make_propagate(False)


def _edge_dot_body(out_tab, rowdp_hbm, coldp_hbm, res_hbm,
                   iav, ibv, a0, b0, a1, b1, p0, p1,
                   g0, g1, q0, q1):
    c = lax.axis_index("c")
    s = lax.axis_index("s")
    wid = s * NC + c
    woff = wid * EPW
    abuf = [a0, a1]
    bbuf = [b0, b1]
    pbuf = [p0, p1]
    gsem = [g0, g1]
    psem = [q0, q1]

    da = pltpu.async_copy(rowdp_hbm.at[wid], iav, g0)
    db = pltpu.async_copy(coldp_hbm.at[wid], ibv, g1)
    da.wait()
    db.wait()

    def start_g(j, k):
        pltpu.async_copy(out_tab.at[iav.at[j]], abuf[k], gsem[k])
        pltpu.async_copy(out_tab.at[ibv.at[j]], bbuf[k], gsem[k])

    def wait_g(j, k):
        pltpu.make_async_copy(out_tab.at[iav.at[j]], abuf[k],
                              gsem[k]).wait()
        pltpu.make_async_copy(out_tab.at[ibv.at[j]], bbuf[k],
                              gsem[k]).wait()

    def start_p(j, k):
        @pl.when(j < NBD - 1)
        def _():
            pltpu.async_copy(pbuf[k],
                             res_hbm.at[pl.ds(woff + j * EB, EB)], psem[k])

        @pl.when(j == NBD - 1)
        def _():
            pltpu.async_copy(pbuf[k].at[pl.ds(0, DLAST)],
                             res_hbm.at[pl.ds(woff + j * EB, DLAST)],
                             psem[k])

    def wait_p(j, k):
        pltpu.make_async_copy(
            pbuf[k], res_hbm.at[pl.ds(woff + j * EB, EB)], psem[k]).wait()

    def compute(k):
        aref = abuf[k]
        bref = bbuf[k]
        pref = pbuf[k]

        def dot1(e, _):
            p = aref[e, pl.ds(0, L)] * bref[e, pl.ds(0, L)]
            for q in range(1, D // L):
                p = p + aref[e, pl.ds(q * L, L)] * bref[e, pl.ds(q * L, L)]
            pref[e] = p
            return 0

        lax.fori_loop(0, EB, dot1, 0)

    def group(i, _):
        for k in range(2):
            j = 2 * i + k

            @pl.when(j >= 2)
            def _():
                wait_p(jnp.maximum(j - 2, 0), k)

            start_g(j, k)
        for k in range(2):
            j = 2 * i + k
            wait_g(j, k)
            compute(k)
            start_p(j, k)
        return 0

    lax.fori_loop(0, NBD // 2, group, 0)
    wait_p(NBD - 2, 0)
    pltpu.make_async_copy(
        pbuf[1].at[pl.ds(0, DLAST)],
        res_hbm.at[pl.ds(woff + (NBD - 1) * EB, DLAST)], psem[1]).wait()


_edge_dot = pl.kernel(
    _edge_dot_body,
    out_type=jax.ShapeDtypeStruct((E, L), jnp.float32),
    mesh=_mesh,
    scratch_types=[
        pltpu.VMEM((NBD, EB), jnp.int32),
        pltpu.VMEM((NBD, EB), jnp.int32),
        pltpu.VMEM((EB, D), jnp.float32),
        pltpu.VMEM((EB, D), jnp.float32),
        pltpu.VMEM((EB, D), jnp.float32),
        pltpu.VMEM((EB, D), jnp.float32),
        pltpu.VMEM((EB, L), jnp.float32),
        pltpu.VMEM((EB, L), jnp.float32),
        pltpu.SemaphoreType.DMA,
        pltpu.SemaphoreType.DMA,
        pltpu.SemaphoreType.DMA,
        pltpu.SemaphoreType.DMA,
    ],
    compiler_params=_sc_params)


_NBLK = 2000


def _init_body(deg_ref, emb_ref, a0_ref, dis_ref, y_ref, out_ref):
    deg = deg_ref[0] + deg_ref[1]
    dis = jnp.where(deg > 0, lax.rsqrt(deg), 0.0)
    dis_ref[...] = dis
    e = emb_ref[...]
    y_ref[0] = dis * e[:, :DH]
    y_ref[1] = dis * e[:, DH:]
    out_ref[...] = a0_ref[0, 0] * e


def _tc_init(deg2, emb, alpha0):
    return pl.pallas_call(
        _init_body,
        grid=(N_NODES // _NBLK,),
        in_specs=[pl.BlockSpec((NC, _NBLK, DH), lambda i: (0, i, 0)),
                  pl.BlockSpec((_NBLK, D), lambda i: (i, 0)),
                  pl.BlockSpec(memory_space=pltpu.SMEM)],
        out_specs=[pl.BlockSpec((_NBLK, DH), lambda i: (i, 0)),
                   pl.BlockSpec((NC, _NBLK, DH), lambda i: (0, i, 0)),
                   pl.BlockSpec((_NBLK, D), lambda i: (i, 0))],
        out_shape=[jax.ShapeDtypeStruct((N_NODES, DH), jnp.float32),
                   jax.ShapeDtypeStruct((NC, N_NODES, DH), jnp.float32),
                   jax.ShapeDtypeStruct((N_NODES, D), jnp.float32)],
    )(deg2, emb, alpha0.reshape(1, 1))


def _layer_body(acc_ref, dis_ref, out_ref, al_ref, out2_ref, y_ref):
    dis = dis_ref[...]
    al = al_ref[0, 0]
    x0 = dis * acc_ref[0]
    x1 = dis * acc_ref[1]
    o = out_ref[...]
    out2_ref[:, :DH] = o[:, :DH] + al * x0
    out2_ref[:, DH:] = o[:, DH:] + al * x1
    y_ref[0] = dis * x0
    y_ref[1] = dis * x1


def _tc_layer(acc2, dis, out, alpha_l):
    return pl.pallas_call(
        _layer_body,
        grid=(N_NODES // _NBLK,),
        in_specs=[pl.BlockSpec((NC, _NBLK, DH), lambda i: (0, i, 0)),
                  pl.BlockSpec((_NBLK, DH), lambda i: (i, 0)),
                  pl.BlockSpec((_NBLK, D), lambda i: (i, 0)),
                  pl.BlockSpec(memory_space=pltpu.SMEM)],
        out_specs=[pl.BlockSpec((_NBLK, D), lambda i: (i, 0)),
                   pl.BlockSpec((NC, _NBLK, DH), lambda i: (0, i, 0))],
        out_shape=[jax.ShapeDtypeStruct((N_NODES, D), jnp.float32),
                   jax.ShapeDtypeStruct((NC, N_NODES, DH), jnp.float32)],
    )(acc2, dis, out, alpha_l.reshape(1, 1))


_RBLK = 4000


def _reduce_body(p_ref, res_ref):
    res_ref[...] = jnp.sum(p_ref[...], axis=1, keepdims=True)


def _tc_reduce(p):
    res = pl.pallas_call(
        _reduce_body,
        grid=(E // _RBLK,),
        in_specs=[pl.BlockSpec((_RBLK, L), lambda i: (i, 0))],
        out_specs=pl.BlockSpec((_RBLK, 1), lambda i: (i, 0)),
        out_shape=jax.ShapeDtypeStruct((E, 1), jnp.float32),
    )(p)
    return res.reshape(E)


def kernel(edge_index, emb_weight, alpha):
    row = edge_index[0]
    col = edge_index[1]

    padg = EPT_G - E // NS
    rowp = jnp.pad(row.reshape(NS, E // NS), ((0, 0), (0, padg)),
                   constant_values=0)
    rowp2 = jnp.concatenate([rowp, rowp + N_NODES], axis=0).reshape(-1)
    colp = jnp.pad(col.reshape(NS, E // NS), ((0, 0), (0, padg)),
                   constant_values=TRASH)
    colp2 = jnp.concatenate([colp, colp], axis=0).reshape(-1)

    pado = EPT_O - E // (NC * NS)
    cold = jnp.pad(col.reshape(NC * NS, E // (NC * NS)),
                   ((0, 0), (0, pado)), constant_values=TRASH).reshape(-1)

    padd = NBD * EB - EPW
    rowdp = jnp.pad(row.reshape(NC * NS, EPW), ((0, 0), (0, padd)),
                    constant_values=0).reshape(NC * NS, NBD, EB)
    coldp = jnp.pad(col.reshape(NC * NS, EPW), ((0, 0), (0, padd)),
                    constant_values=0).reshape(NC * NS, NBD, EB)

    deg2 = _prop_ones(cold)
    dis, y2, out = _tc_init(deg2.reshape(NC, N_NODES, DH), emb_weight,
                            alpha[0])

    for l in range(NUM_LAYERS):
        acc2 = _prop_gather(y2.reshape(NC * N_NODES, DH), rowp2, colp2)
        out, y2 = _tc_layer(acc2.reshape(NC, N_NODES, DH), dis, out,
                            alpha[l + 1])

    return _tc_reduce(_edge_dot(out, rowdp, coldp))
